# baseline (speedup 1.0000x reference)
import functools

import jax
import jax.numpy as jnp
from jax import lax
from jax.experimental import pallas as pl
from jax.experimental.pallas import tpu as pltpu
from jax.experimental.pallas import tpu_sc as plsc

H = 256
W = 256
D = 768
LANES = 16
NC = 2
NS = 16
NW = NC * NS
RPW = H // NW
JC = 32
NJ = W // JC
LG = D // LANES

_mesh = plsc.VectorSubcoreMesh(core_axis_name="c", subcore_axis_name="s")


@functools.partial(
    pl.kernel,
    mesh=_mesh,
    out_type=jax.ShapeDtypeStruct((H * W, D), jnp.float32),
    scratch_types=[
        pltpu.VMEM((RPW,), jnp.int32),
        pltpu.VMEM((JC,), jnp.int32),
        pltpu.VMEM((RPW, D), jnp.float32),
        pltpu.VMEM((JC, D), jnp.float32),
        pltpu.VMEM((JC, D), jnp.float32),
        pltpu.SemaphoreType.DMA,
    ],
)
def _sc_embed(rows_hbm, cols_hbm, row_table, col_table, out_hbm,
              ridx_v, cidx_v, rowe_v, cole_v, outb_v, sem):
    wid = lax.axis_index("s") * NC + lax.axis_index("c")
    rbase = wid * RPW
    pltpu.sync_copy(rows_hbm.at[pl.ds(rbase, RPW)], ridx_v)
    pltpu.async_copy(row_table.at[ridx_v], rowe_v, sem).wait()

    def chunk_body(cj, _):
        j0 = cj * JC
        pltpu.sync_copy(cols_hbm.at[pl.ds(j0, JC)], cidx_v)
        pltpu.async_copy(col_table.at[cidx_v], cole_v, sem).wait()

        def i_body(il, _):
            rvs = [rowe_v[il, pl.ds(g * LANES, LANES)] for g in range(LG)]

            def r_body(r, _):
                for g in range(LG):
                    sl = pl.ds(g * LANES, LANES)
                    outb_v[r, sl] = cole_v[r, sl] + rvs[g]
                return 0

            lax.fori_loop(0, JC, r_body, 0)
            out_start = (rbase + il) * W + j0
            pltpu.sync_copy(outb_v, out_hbm.at[pl.ds(out_start, JC)])
            return 0

        lax.fori_loop(0, RPW, i_body, 0)
        return 0

    lax.fori_loop(0, NJ, chunk_body, 0)


def kernel(grid_size, row_table, col_table):
    gh = jnp.asarray(grid_size[0], jnp.int32)
    gw = jnp.asarray(grid_size[1], jnp.int32)
    rows = jnp.minimum(jnp.arange(H, dtype=jnp.int32), gh - 1)
    cols = jnp.minimum(jnp.arange(W, dtype=jnp.int32), gw - 1)
    return _sc_embed(rows, cols, row_table, col_table)
